# baseline (speedup 1.0000x reference)
import jax
import jax.numpy as jnp
from jax import lax
from jax.experimental import pallas as pl
from jax.experimental.pallas import tpu as pltpu

VT = 1024


def _mlp_body(e_ref, w1_ref, b1_ref, w2_ref, b2_ref, out_ref, h_ref):
    @pl.when(pl.program_id(0) == 0)
    def _():
        e = e_ref[...].astype(jnp.bfloat16)
        w1 = w1_ref[...].astype(jnp.bfloat16)
        h = lax.dot_general(e, w1, (((1,), (1,)), ((), ())),
                            preferred_element_type=jnp.float32)
        h = h + b1_ref[...][None, :]
        h_ref[...] = jnp.maximum(h, 0.0).astype(jnp.bfloat16)

    w2 = w2_ref[...].astype(jnp.bfloat16)
    out = lax.dot_general(h_ref[...], w2, (((1,), (1,)), ((), ())),
                          preferred_element_type=jnp.float32)
    out_ref[...] = out + b2_ref[...][None, :]


def _mlp(e, W1, b1, W2, b2):
    B = e.shape[0]
    HID = W1.shape[0]
    VOCAB = W2.shape[0]
    grid = (pl.cdiv(VOCAB, VT),)
    return pl.pallas_call(
        _mlp_body,
        grid=grid,
        in_specs=[
            pl.BlockSpec((B, e.shape[1]), lambda i: (0, 0)),
            pl.BlockSpec((HID, W1.shape[1]), lambda i: (0, 0)),
            pl.BlockSpec((HID,), lambda i: (0,)),
            pl.BlockSpec((VT, HID), lambda i: (i, 0)),
            pl.BlockSpec((VT,), lambda i: (i,)),
        ],
        out_specs=pl.BlockSpec((B, VT), lambda i: (0, i)),
        out_shape=jax.ShapeDtypeStruct((B, VOCAB), jnp.float32),
        scratch_shapes=[pltpu.VMEM((B, HID), jnp.bfloat16)],
        compiler_params=pltpu.CompilerParams(
            dimension_semantics=("arbitrary",),
        ),
    )(e, W1, b1, W2, b2)


@jax.jit
def kernel(x, emb, W1, b1, W2, b2):
    e = jnp.take(emb, x.reshape(-1), axis=0)
    e = e.reshape(x.shape[0], -1)
    return _mlp(e, W1, b1, W2, b2)


# speedup vs baseline: 1.0326x; 1.0326x over previous
import jax
import jax.numpy as jnp
from jax import lax
from jax.experimental import pallas as pl
from jax.experimental.pallas import tpu as pltpu
from jax.experimental.pallas import tpu_sc as plsc

VT = 2048
GATHER_WINDOW = 256


def _sc_gather(emb, idx_flat):
    n = idx_flat.shape[0]
    emb_dim = emb.shape[1]
    idx2d = idx_flat.reshape(1, n)

    mesh = plsc.VectorSubcoreMesh(core_axis_name="c", subcore_axis_name="s")

    @pl.kernel(
        out_type=jax.ShapeDtypeStruct((n, emb_dim), emb.dtype),
        mesh=mesh,
        compiler_params=pltpu.CompilerParams(use_tc_tiling_on_sc=False),
    )
    def gather_kernel(emb_hbm, i_hbm, o_hbm):
        def body(i_vmem, o_vmem):
            pltpu.sync_copy(emb_hbm.at[i_vmem.at[0]], o_vmem)

        pltpu.emit_pipeline(
            body,
            grid=(n // GATHER_WINDOW,),
            in_specs=[pl.BlockSpec((1, GATHER_WINDOW), index_map=lambda i: (0, i))],
            out_specs=[pl.BlockSpec((GATHER_WINDOW, emb_dim),
                                    index_map=lambda i: (i, 0))],
            core_axis_name=("c", "s"),
            dimension_semantics=(pltpu.PARALLEL,),
        )(i_hbm, o_hbm)

    return gather_kernel(emb, idx2d)


def _mlp_body(e_ref, w1_ref, b1_ref, w2_ref, b2_ref, out_ref, h_ref):
    @pl.when(pl.program_id(0) == 0)
    def _():
        e = e_ref[...].astype(jnp.bfloat16)
        w1 = w1_ref[...].astype(jnp.bfloat16)
        h = lax.dot_general(e, w1, (((1,), (1,)), ((), ())),
                            preferred_element_type=jnp.float32)
        h = h + b1_ref[...][None, :]
        h_ref[...] = jnp.maximum(h, 0.0).astype(jnp.bfloat16)

    w2 = w2_ref[...].astype(jnp.bfloat16)
    out = lax.dot_general(h_ref[...], w2, (((1,), (1,)), ((), ())),
                          preferred_element_type=jnp.float32)
    out_ref[...] = out + b2_ref[...][None, :]


def _mlp(e, W1, b1, W2, b2):
    B = e.shape[0]
    HID = W1.shape[0]
    VOCAB = W2.shape[0]
    grid = (pl.cdiv(VOCAB, VT),)
    return pl.pallas_call(
        _mlp_body,
        grid=grid,
        in_specs=[
            pl.BlockSpec((B, e.shape[1]), lambda i: (0, 0)),
            pl.BlockSpec((HID, W1.shape[1]), lambda i: (0, 0)),
            pl.BlockSpec((HID,), lambda i: (0,)),
            pl.BlockSpec((VT, HID), lambda i: (i, 0)),
            pl.BlockSpec((VT,), lambda i: (i,)),
        ],
        out_specs=pl.BlockSpec((B, VT), lambda i: (0, i)),
        out_shape=jax.ShapeDtypeStruct((B, VOCAB), jnp.float32),
        scratch_shapes=[pltpu.VMEM((B, HID), jnp.bfloat16)],
        compiler_params=pltpu.CompilerParams(
            dimension_semantics=("arbitrary",),
        ),
    )(e, W1, b1, W2, b2)


@jax.jit
def kernel(x, emb, W1, b1, W2, b2):
    B = x.shape[0]
    e_rows = _sc_gather(emb, x.reshape(-1))
    e = e_rows.reshape(B, -1)
    return _mlp(e, W1, b1, W2, b2)


# speedup vs baseline: 1.0327x; 1.0001x over previous
import functools

import jax
import jax.numpy as jnp
from jax import lax
import jax.experimental.layout as jlayout
from jax.sharding import SingleDeviceSharding
from jax.experimental import pallas as pl
from jax.experimental.pallas import tpu as pltpu
from jax.experimental.pallas import tpu_sc as plsc

VT = 2048
GATHER_WINDOW = 256


def _sc_gather(emb, idx_flat):
    n = idx_flat.shape[0]
    emb_dim = emb.shape[1]
    idx2d = idx_flat.reshape(1, n)

    mesh = plsc.VectorSubcoreMesh(core_axis_name="c", subcore_axis_name="s")

    @pl.kernel(
        out_type=jax.ShapeDtypeStruct((n, emb_dim), emb.dtype),
        mesh=mesh,
        compiler_params=pltpu.CompilerParams(use_tc_tiling_on_sc=False),
    )
    def gather_kernel(emb_hbm, i_hbm, o_hbm):
        def body(i_vmem, o_vmem):
            pltpu.sync_copy(emb_hbm.at[i_vmem.at[0]], o_vmem)

        pltpu.emit_pipeline(
            body,
            grid=(n // GATHER_WINDOW,),
            in_specs=[pl.BlockSpec((1, GATHER_WINDOW), index_map=lambda i: (0, i))],
            out_specs=[pl.BlockSpec((GATHER_WINDOW, emb_dim),
                                    index_map=lambda i: (i, 0))],
            core_axis_name=("c", "s"),
            dimension_semantics=(pltpu.PARALLEL,),
        )(i_hbm, o_hbm)

    return gather_kernel(emb, idx2d)


def _mlp_body(e_ref, w1_ref, b1_ref, w2_ref, b2_ref, out_ref, h_ref):
    @pl.when(pl.program_id(0) == 0)
    def _():
        e = e_ref[...].astype(jnp.bfloat16)
        w1 = w1_ref[...].astype(jnp.bfloat16)
        h = lax.dot_general(e, w1, (((1,), (1,)), ((), ())),
                            preferred_element_type=jnp.float32)
        h = h + b1_ref[...][None, :]
        h_ref[...] = jnp.maximum(h, 0.0).astype(jnp.bfloat16)

    w2 = w2_ref[...].astype(jnp.bfloat16)
    out = lax.dot_general(h_ref[...], w2, (((1,), (1,)), ((), ())),
                          preferred_element_type=jnp.float32)
    out_ref[...] = out + b2_ref[...][None, :]


def _mlp(e, W1, b1, W2, b2):
    B = e.shape[0]
    HID = W1.shape[0]
    VOCAB = W2.shape[0]
    grid = (pl.cdiv(VOCAB, VT),)
    return pl.pallas_call(
        _mlp_body,
        grid=grid,
        in_specs=[
            pl.BlockSpec((B, e.shape[1]), lambda i: (0, 0)),
            pl.BlockSpec((HID, W1.shape[1]), lambda i: (0, 0)),
            pl.BlockSpec((HID,), lambda i: (0,)),
            pl.BlockSpec((VT, HID), lambda i: (i, 0)),
            pl.BlockSpec((VT,), lambda i: (i,)),
        ],
        out_specs=pl.BlockSpec((B, VT), lambda i: (0, i)),
        out_shape=jax.ShapeDtypeStruct((B, VOCAB), jnp.float32),
        scratch_shapes=[pltpu.VMEM((B, HID), jnp.bfloat16)],
        compiler_params=pltpu.CompilerParams(
            dimension_semantics=("arbitrary",),
        ),
    )(e, W1, b1, W2, b2)


def _impl(x, emb, W1, b1, W2, b2):
    B = x.shape[0]
    e_rows = _sc_gather(emb, x.reshape(-1))
    e = e_rows.reshape(B, -1)
    return _mlp(e, W1, b1, W2, b2)


@functools.lru_cache(maxsize=None)
def _jitted(dev):
    fmt = jlayout.Format(jlayout.Layout(major_to_minor=(1, 0), tiling=()),
                         SingleDeviceSharding(dev))
    return jax.jit(_impl, out_shardings=fmt)


def kernel(x, emb, W1, b1, W2, b2):
    try:
        dev = next(iter(x.devices()))
    except Exception:
        dev = jax.devices()[0]
    return _jitted(dev)(x, emb, W1, b1, W2, b2)


# speedup vs baseline: 2.2836x; 2.2112x over previous
import jax
import jax.numpy as jnp
from jax import lax
from jax.experimental import pallas as pl
from jax.experimental.pallas import tpu as pltpu
from jax.experimental.pallas import tpu_sc as plsc

VT = 2048
GATHER_WINDOW = 256


def _sc_gather(emb, idx_flat):
    n = idx_flat.shape[0]
    emb_dim = emb.shape[1]
    idx2d = idx_flat.reshape(1, n)

    mesh = plsc.VectorSubcoreMesh(core_axis_name="c", subcore_axis_name="s")

    @pl.kernel(
        out_type=jax.ShapeDtypeStruct((n, emb_dim), emb.dtype),
        mesh=mesh,
        compiler_params=pltpu.CompilerParams(use_tc_tiling_on_sc=False),
    )
    def gather_kernel(emb_hbm, i_hbm, o_hbm):
        def body(i_vmem, o_vmem):
            pltpu.sync_copy(emb_hbm.at[i_vmem.at[0]], o_vmem)

        pltpu.emit_pipeline(
            body,
            grid=(n // GATHER_WINDOW,),
            in_specs=[pl.BlockSpec((1, GATHER_WINDOW), index_map=lambda i: (0, i))],
            out_specs=[pl.BlockSpec((GATHER_WINDOW, emb_dim),
                                    index_map=lambda i: (i, 0))],
            core_axis_name=("c", "s"),
            dimension_semantics=(pltpu.PARALLEL,),
        )(i_hbm, o_hbm)

    return gather_kernel(emb, idx2d)


def _mlp_body(e_ref, w1t_ref, b1_ref, w2_ref, b2_ref, out_ref, h_ref):
    @pl.when(pl.program_id(0) == 0)
    def _():
        e = e_ref[...].astype(jnp.bfloat16)
        w1t = w1t_ref[...].astype(jnp.bfloat16)
        h = lax.dot_general(e, w1t, (((1,), (0,)), ((), ())),
                            preferred_element_type=jnp.float32)
        h = h + b1_ref[...][None, :]
        h_ref[...] = jnp.maximum(h, 0.0).astype(jnp.bfloat16)

    w2 = w2_ref[...].astype(jnp.bfloat16)
    out = lax.dot_general(w2, h_ref[...], (((1,), (1,)), ((), ())),
                          preferred_element_type=jnp.float32)
    out_ref[...] = out + b2_ref[...][:, None]


def _mlp_t(e, W1t, b1, W2, b2):
    B = e.shape[0]
    HID = W1t.shape[1]
    VOCAB = W2.shape[0]
    grid = (pl.cdiv(VOCAB, VT),)
    return pl.pallas_call(
        _mlp_body,
        grid=grid,
        in_specs=[
            pl.BlockSpec((B, e.shape[1]), lambda i: (0, 0)),
            pl.BlockSpec((W1t.shape[0], HID), lambda i: (0, 0)),
            pl.BlockSpec((HID,), lambda i: (0,)),
            pl.BlockSpec((VT, HID), lambda i: (i, 0)),
            pl.BlockSpec((VT,), lambda i: (i,)),
        ],
        out_specs=pl.BlockSpec((VT, B), lambda i: (i, 0)),
        out_shape=jax.ShapeDtypeStruct((VOCAB, B), jnp.float32),
        scratch_shapes=[pltpu.VMEM((B, HID), jnp.bfloat16)],
        compiler_params=pltpu.CompilerParams(
            dimension_semantics=("arbitrary",),
        ),
    )(e, W1t, b1, W2, b2)


@jax.jit
def kernel(x, emb, W1, b1, W2, b2):
    B = x.shape[0]
    e_rows = _sc_gather(emb, x.reshape(-1))
    e = e_rows.reshape(B, -1)
    out_t = _mlp_t(e, W1.T, b1, W2, b2)
    return out_t.T
